# baseline (speedup 1.0000x reference)
import jax
import jax.numpy as jnp
from jax import lax
from jax.experimental import pallas as pl

_D = 1024
_WDIM = 20
_VOCAB = 8
_BS = 1024


def _body(starts_ref, ends_ref, emb_ref, table_ref, out_ref):
    out_ref[:, :_D] = emb_ref[...]
    widths = ends_ref[0] - starts_ref[0]
    onehot = (widths == lax.broadcasted_iota(jnp.int32, (_BS, _VOCAB), 1))
    wemb = jnp.dot(onehot.astype(jnp.float32), table_ref[...],
                   preferred_element_type=jnp.float32)
    out_ref[:, _D:] = wemb


def kernel(spans, span_embeddings, width_table):
    B, S, D = span_embeddings.shape
    rows = B * S
    nb = rows // _BS
    starts = spans[..., 0].astype(jnp.int32).reshape(nb, _BS, 1)
    ends = spans[..., 1].astype(jnp.int32).reshape(nb, _BS, 1)
    emb = span_embeddings.reshape(rows, D)
    out = pl.pallas_call(
        _body,
        grid=(nb,),
        in_specs=[
            pl.BlockSpec((1, _BS, 1), lambda i: (i, 0, 0)),
            pl.BlockSpec((1, _BS, 1), lambda i: (i, 0, 0)),
            pl.BlockSpec((_BS, D), lambda i: (i, 0)),
            pl.BlockSpec((_VOCAB, _WDIM), lambda i: (0, 0)),
        ],
        out_specs=pl.BlockSpec((_BS, D + _WDIM), lambda i: (i, 0)),
        out_shape=jax.ShapeDtypeStruct((rows, D + _WDIM), jnp.float32),
    )(starts, ends, emb, width_table)
    return out.reshape(B, S, D + _WDIM)
